# Initial kernel scaffold; baseline (speedup 1.0000x reference)
import jax
import jax.numpy as jnp
from jax.experimental import pallas as pl


def kernel(input_ids, token_type_ids, token_table, W_proj, pos_table, type_table, gamma, beta):
    raise NotImplementedError("write your pallas kernel here")


# speedup vs baseline: 3.9193x; 3.9193x over previous
import functools

import jax
import jax.numpy as jnp
from jax import lax
from jax.experimental import pallas as pl
from jax.experimental.pallas import tpu as pltpu
from jax.experimental.pallas import tpu_sc as plsc

NC = 2
NS = 16
NW = NC * NS
IDX_CHUNK = 128


def _sc_gather(table, idx_flat):
    tot = idx_flat.shape[0]
    D = table.shape[1]
    b_per_w = tot // NW
    n_chunks = b_per_w // IDX_CHUNK
    mesh = plsc.VectorSubcoreMesh(
        core_axis_name="c", subcore_axis_name="s",
        num_cores=NC, num_subcores=NS,
    )

    @functools.partial(
        pl.kernel,
        mesh=mesh,
        out_type=jax.ShapeDtypeStruct((tot, D), jnp.float32),
        scratch_types=[
            pltpu.VMEM((n_chunks, IDX_CHUNK), jnp.int32),
            pltpu.VMEM((b_per_w, D), jnp.float32),
            pltpu.SemaphoreType.DMA,
        ],
    )
    def gather_k(table_hbm, idx_hbm, out_hbm, idx_v, rows_v, sem):
        wid = lax.axis_index("s") * NC + lax.axis_index("c")
        base = wid * b_per_w
        pltpu.sync_copy(idx_hbm.at[wid], idx_v)
        copies = []
        for j in range(n_chunks):
            copies.append(
                pltpu.async_copy(
                    table_hbm.at[idx_v.at[j]],
                    rows_v.at[pl.ds(j * IDX_CHUNK, IDX_CHUNK)],
                    sem,
                )
            )
        for c in copies:
            c.wait()
        pltpu.sync_copy(rows_v, out_hbm.at[pl.ds(base, b_per_w)])

    return gather_k(table, idx_flat.reshape(NW, n_chunks, IDX_CHUNK))


def _tc_fused(xg, w_t, ttf, pos_table, type_table, gamma2, beta2, B, S, H, T):
    E = xg.shape[1]
    SB = S // T

    def body(x_ref, w_ref, tt_ref, pos_ref, type_ref, g_ref, b_ref, o_ref):
        h = jnp.dot(x_ref[...], w_ref[...], preferred_element_type=jnp.float32)
        h = h + pos_ref[...]
        t0 = type_ref[0:1, :]
        dt = type_ref[1:2, :] - t0
        h = h + t0 + tt_ref[...] * dt
        mu = jnp.mean(h, axis=1, keepdims=True)
        d = h - mu
        var = jnp.mean(d * d, axis=1, keepdims=True)
        o_ref[...] = d * lax.rsqrt(var + 1e-5) * g_ref[...] + b_ref[...]

    return pl.pallas_call(
        body,
        grid=(SB, B),
        in_specs=[
            pl.BlockSpec((T, E), lambda s, b: (b * SB + s, 0)),
            pl.BlockSpec((E, H), lambda s, b: (0, 0)),
            pl.BlockSpec((T, 1), lambda s, b: (b * SB + s, 0)),
            pl.BlockSpec((T, H), lambda s, b: (s, 0)),
            pl.BlockSpec((2, H), lambda s, b: (0, 0)),
            pl.BlockSpec((1, H), lambda s, b: (0, 0)),
            pl.BlockSpec((1, H), lambda s, b: (0, 0)),
        ],
        out_specs=pl.BlockSpec((T, H), lambda s, b: (b * SB + s, 0)),
        out_shape=jax.ShapeDtypeStruct((B * S, H), jnp.float32),
    )(xg, w_t, ttf, pos_table, type_table, gamma2, beta2)


def kernel(input_ids, token_type_ids, token_table, W_proj, pos_table,
           type_table, gamma, beta):
    B, S = input_ids.shape
    H, E = W_proj.shape
    tot = B * S
    T = 256

    idx = input_ids.reshape(tot).astype(jnp.int32)
    xg = _sc_gather(token_table, idx)

    ttf = token_type_ids.reshape(tot, 1).astype(jnp.float32)
    w_t = W_proj.T
    out = _tc_fused(xg, w_t, ttf, pos_table[:S], type_table,
                    gamma.reshape(1, H), beta.reshape(1, H), B, S, H, T)
    return out.reshape(B, S, H)


# speedup vs baseline: 4.4730x; 1.1413x over previous
import functools

import jax
import jax.numpy as jnp
from jax import lax
from jax.experimental import pallas as pl
from jax.experimental.pallas import tpu as pltpu
from jax.experimental.pallas import tpu_sc as plsc

NC = 2
NS = 16
NW = NC * NS
IDX_CHUNK = 128


def _sc_gather(table, idx_flat):
    tot = idx_flat.shape[0]
    D = table.shape[1]
    b_per_w = tot // NW
    n_chunks = b_per_w // IDX_CHUNK
    mesh = plsc.VectorSubcoreMesh(
        core_axis_name="c", subcore_axis_name="s",
        num_cores=NC, num_subcores=NS,
    )

    @functools.partial(
        pl.kernel,
        mesh=mesh,
        out_type=jax.ShapeDtypeStruct((tot, D), jnp.float32),
        scratch_types=[
            pltpu.VMEM((n_chunks, IDX_CHUNK), jnp.int32),
            pltpu.VMEM((b_per_w, D), jnp.float32),
            pltpu.SemaphoreType.DMA,
        ],
    )
    def gather_k(table_hbm, idx_hbm, out_hbm, idx_v, rows_v, sem):
        wid = lax.axis_index("s") * NC + lax.axis_index("c")
        base = wid * b_per_w
        pltpu.sync_copy(idx_hbm.at[wid], idx_v)
        copies = []
        for j in range(n_chunks):
            copies.append(
                pltpu.async_copy(
                    table_hbm.at[idx_v.at[j]],
                    rows_v.at[pl.ds(j * IDX_CHUNK, IDX_CHUNK)],
                    sem,
                )
            )
        for c in copies:
            c.wait()
        pltpu.sync_copy(rows_v, out_hbm.at[pl.ds(base, b_per_w)])

    return gather_k(table, idx_flat.reshape(NW, n_chunks, IDX_CHUNK))


def _tc_fused(xg, w_t, ttf, pos_table, type_table, gamma2, beta2, B, S, H, T):
    E = xg.shape[1]
    SB = S // T

    def body(x_ref, w_ref, tt_ref, pos_ref, type_ref, g_ref, b_ref, o_ref):
        h = jnp.dot(x_ref[...], w_ref[...], preferred_element_type=jnp.float32)
        h = h + pos_ref[...]
        t0 = type_ref[0:1, :]
        dt = type_ref[1:2, :] - t0
        h = h + t0 + tt_ref[...] * dt
        mu = jnp.mean(h, axis=1, keepdims=True)
        d = h - mu
        var = jnp.mean(d * d, axis=1, keepdims=True)
        o_ref[...] = d * lax.rsqrt(var + 1e-5) * g_ref[...] + b_ref[...]

    return pl.pallas_call(
        body,
        grid=(SB, B),
        in_specs=[
            pl.BlockSpec((T, E), lambda s, b: (b * SB + s, 0)),
            pl.BlockSpec((E, H), lambda s, b: (0, 0)),
            pl.BlockSpec((T, 1), lambda s, b: (b * SB + s, 0)),
            pl.BlockSpec((T, H), lambda s, b: (s, 0)),
            pl.BlockSpec((2, H), lambda s, b: (0, 0)),
            pl.BlockSpec((1, H), lambda s, b: (0, 0)),
            pl.BlockSpec((1, H), lambda s, b: (0, 0)),
        ],
        out_specs=pl.BlockSpec((T, H), lambda s, b: (b * SB + s, 0)),
        out_shape=jax.ShapeDtypeStruct((B * S, H), jnp.float32),
    )(xg, w_t, ttf, pos_table, type_table, gamma2, beta2)


def kernel(input_ids, token_type_ids, token_table, W_proj, pos_table,
           type_table, gamma, beta):
    B, S = input_ids.shape
    H, E = W_proj.shape
    tot = B * S
    T = 512

    idx = input_ids.reshape(tot).astype(jnp.int32)
    xg = _sc_gather(token_table, idx)

    ttf = token_type_ids.reshape(tot, 1).astype(jnp.float32)
    w_t = W_proj.T
    out = _tc_fused(xg, w_t, ttf, pos_table[:S], type_table,
                    gamma.reshape(1, H), beta.reshape(1, H), B, S, H, T)
    return out.reshape(B, S, H)


# speedup vs baseline: 4.8037x; 1.0740x over previous
import functools

import jax
import jax.numpy as jnp
from jax import lax
from jax.experimental import pallas as pl
from jax.experimental.pallas import tpu as pltpu
from jax.experimental.pallas import tpu_sc as plsc

NC = 2
NS = 16
NW = NC * NS
IDX_CHUNK = 128


def _sc_gather(table, idx_flat):
    tot = idx_flat.shape[0]
    D = table.shape[1]
    b_per_w = tot // NW
    n_chunks = b_per_w // IDX_CHUNK
    mesh = plsc.VectorSubcoreMesh(
        core_axis_name="c", subcore_axis_name="s",
        num_cores=NC, num_subcores=NS,
    )

    @functools.partial(
        pl.kernel,
        mesh=mesh,
        out_type=jax.ShapeDtypeStruct((tot, D), jnp.float32),
        scratch_types=[
            pltpu.VMEM((n_chunks, IDX_CHUNK), jnp.int32),
            pltpu.VMEM((b_per_w, D), jnp.float32),
            pltpu.SemaphoreType.DMA,
        ],
    )
    def gather_k(table_hbm, idx_hbm, out_hbm, idx_v, rows_v, sem):
        wid = lax.axis_index("s") * NC + lax.axis_index("c")
        base = wid * b_per_w
        pltpu.sync_copy(idx_hbm.at[wid], idx_v)
        copies = []
        for j in range(n_chunks):
            copies.append(
                pltpu.async_copy(
                    table_hbm.at[idx_v.at[j]],
                    rows_v.at[pl.ds(j * IDX_CHUNK, IDX_CHUNK)],
                    sem,
                )
            )
        for c in copies:
            c.wait()
        pltpu.sync_copy(rows_v, out_hbm.at[pl.ds(base, b_per_w)])

    return gather_k(table, idx_flat.reshape(NW, n_chunks, IDX_CHUNK))


def _tc_fused(xg, w_t, ttf, pos_table, type_table, gamma2, beta2, B, S, H, T):
    E = xg.shape[1]
    SB = S // T

    def body(x_ref, w_ref, tt_ref, pos_ref, type_ref, g_ref, b_ref, o_ref):
        h = jnp.dot(x_ref[...], w_ref[...], preferred_element_type=jnp.float32)
        h = h + pos_ref[...]
        t0 = type_ref[0:1, :]
        dt = type_ref[1:2, :] - t0
        h = h + t0 + tt_ref[...] * dt
        mu = jnp.mean(h, axis=1, keepdims=True)
        d = h - mu
        var = jnp.mean(d * d, axis=1, keepdims=True)
        o_ref[...] = d * lax.rsqrt(var + 1e-5) * g_ref[...] + b_ref[...]

    return pl.pallas_call(
        body,
        grid=(SB, B),
        in_specs=[
            pl.BlockSpec((T, E), lambda s, b: (b * SB + s, 0)),
            pl.BlockSpec((E, H), lambda s, b: (0, 0)),
            pl.BlockSpec((T, 1), lambda s, b: (b * SB + s, 0)),
            pl.BlockSpec((T, H), lambda s, b: (s, 0)),
            pl.BlockSpec((2, H), lambda s, b: (0, 0)),
            pl.BlockSpec((1, H), lambda s, b: (0, 0)),
            pl.BlockSpec((1, H), lambda s, b: (0, 0)),
        ],
        out_specs=pl.BlockSpec((T, H), lambda s, b: (b * SB + s, 0)),
        out_shape=jax.ShapeDtypeStruct((B * S, H), jnp.float32),
    )(xg, w_t, ttf, pos_table, type_table, gamma2, beta2)


def kernel(input_ids, token_type_ids, token_table, W_proj, pos_table,
           type_table, gamma, beta):
    B, S = input_ids.shape
    H, E = W_proj.shape
    tot = B * S
    T = 1024

    idx = input_ids.reshape(tot).astype(jnp.int32)
    xg = _sc_gather(token_table, idx)

    ttf = token_type_ids.reshape(tot, 1).astype(jnp.float32)
    w_t = W_proj.T
    out = _tc_fused(xg, w_t, ttf, pos_table[:S], type_table,
                    gamma.reshape(1, H), beta.reshape(1, H), B, S, H, T)
    return out.reshape(B, S, H)


# speedup vs baseline: 4.9499x; 1.0304x over previous
import functools

import jax
import jax.numpy as jnp
from jax import lax
from jax.experimental import pallas as pl
from jax.experimental.pallas import tpu as pltpu
from jax.experimental.pallas import tpu_sc as plsc

NC = 2
NS = 16
NW = NC * NS
IDX_CHUNK = 128


def _sc_gather(table, idx_flat):
    tot = idx_flat.shape[0]
    D = table.shape[1]
    b_per_w = tot // NW
    n_chunks = b_per_w // IDX_CHUNK
    mesh = plsc.VectorSubcoreMesh(
        core_axis_name="c", subcore_axis_name="s",
        num_cores=NC, num_subcores=NS,
    )

    @functools.partial(
        pl.kernel,
        mesh=mesh,
        out_type=jax.ShapeDtypeStruct((tot, D), jnp.float32),
        scratch_types=[
            pltpu.VMEM((n_chunks, IDX_CHUNK), jnp.int32),
            pltpu.VMEM((b_per_w, D), jnp.float32),
            pltpu.SemaphoreType.DMA,
        ],
    )
    def gather_k(table_hbm, idx_hbm, out_hbm, idx_v, rows_v, sem):
        wid = lax.axis_index("s") * NC + lax.axis_index("c")
        base = wid * b_per_w
        pltpu.sync_copy(idx_hbm.at[wid], idx_v)
        copies = []
        for j in range(n_chunks):
            copies.append(
                pltpu.async_copy(
                    table_hbm.at[idx_v.at[j]],
                    rows_v.at[pl.ds(j * IDX_CHUNK, IDX_CHUNK)],
                    sem,
                )
            )
        for c in copies:
            c.wait()
        pltpu.sync_copy(rows_v, out_hbm.at[pl.ds(base, b_per_w)])

    return gather_k(table, idx_flat.reshape(NW, n_chunks, IDX_CHUNK))


def _tc_fused(xg, w_t, ttf, pos_table, type_table, gamma2, beta2, B, S, H, T):
    E = xg.shape[1]
    SB = S // T

    def body(x_ref, w_ref, tt_ref, pos_ref, type_ref, g_ref, b_ref, o_ref):
        h = jnp.dot(x_ref[...], w_ref[...], preferred_element_type=jnp.float32)
        h = h + pos_ref[...]
        t0 = type_ref[0:1, :]
        dt = type_ref[1:2, :] - t0
        h = h + t0 + tt_ref[...] * dt
        mu = jnp.mean(h, axis=1, keepdims=True)
        d = h - mu
        var = jnp.mean(d * d, axis=1, keepdims=True)
        o_ref[...] = d * lax.rsqrt(var + 1e-5) * g_ref[...] + b_ref[...]

    return pl.pallas_call(
        body,
        grid=(SB, B),
        in_specs=[
            pl.BlockSpec((T, E), lambda s, b: (b * SB + s, 0)),
            pl.BlockSpec((E, H), lambda s, b: (0, 0)),
            pl.BlockSpec((T, 1), lambda s, b: (b * SB + s, 0)),
            pl.BlockSpec((T, H), lambda s, b: (s, 0)),
            pl.BlockSpec((2, H), lambda s, b: (0, 0)),
            pl.BlockSpec((1, H), lambda s, b: (0, 0)),
            pl.BlockSpec((1, H), lambda s, b: (0, 0)),
        ],
        out_specs=pl.BlockSpec((T, H), lambda s, b: (b * SB + s, 0)),
        out_shape=jax.ShapeDtypeStruct((B * S, H), jnp.float32),
    )(xg, w_t, ttf, pos_table, type_table, gamma2, beta2)


def kernel(input_ids, token_type_ids, token_table, W_proj, pos_table,
           type_table, gamma, beta):
    B, S = input_ids.shape
    H, E = W_proj.shape
    tot = B * S
    T = 2048

    idx = input_ids.reshape(tot).astype(jnp.int32)
    xg = _sc_gather(token_table, idx)

    ttf = token_type_ids.reshape(tot, 1).astype(jnp.float32)
    w_t = W_proj.T
    out = _tc_fused(xg, w_t, ttf, pos_table[:S], type_table,
                    gamma.reshape(1, H), beta.reshape(1, H), B, S, H, T)
    return out.reshape(B, S, H)
